# baseline (speedup 1.0000x reference)
import jax
import jax.numpy as jnp
import numpy as np
from jax.experimental import pallas as pl
from jax.experimental.pallas import tpu as pltpu

_N = 20000
_NPAD = 20480
_ROWS = _NPAD // 128
_G = 64
_POS = 32
_NEG = 96
_K = _POS + _NEG

_NEG_INF = np.float32(-np.inf)
_I32_MIN = np.int32(-(2 ** 31))
_I32_MAX = np.int32(2 ** 31 - 1)

_DN = (((1,), (0,)), ((), ()))
_DT = (((0,), (0,)), ((), ()))


def _orderkey(x):
    b = jax.lax.bitcast_convert_type(x, jnp.int32)
    return jnp.where(b < 0, b ^ jnp.int32(0x7FFFFFFF), b)


def _sum11(x):
    return jnp.sum(jnp.sum(x, axis=0, keepdims=True), axis=1, keepdims=True)


def _mm(a, b):
    return jax.lax.dot_general(a, b, _DN, preferred_element_type=jnp.float32,
                               precision=jax.lax.Precision.HIGHEST)


def _body(boxes_tr_ref, gt_ref, gt4_ref,
          roi_ref, gtn_ref, label_ref, loc_ref,
          kp_ref, kn_ref, ga_ref):
    b0 = boxes_tr_ref[0]
    b1 = boxes_tr_ref[1]
    b2 = boxes_tr_ref[2]
    b3 = boxes_tr_ref[3]
    area = (b2 - b0) * (b3 - b1)

    def iou_step(g, carry):
        mi, ga = carry
        g0 = gt_ref[0, g]
        g1 = gt_ref[1, g]
        g2 = gt_ref[2, g]
        g3 = gt_ref[3, g]
        ty = jnp.maximum(b0, g0)
        tx = jnp.maximum(b1, g1)
        by = jnp.minimum(b2, g2)
        bx = jnp.minimum(b3, g3)
        inter = ((by - ty) * (bx - tx)) * jnp.where(
            (ty < by) & (tx < bx), jnp.float32(1.0), jnp.float32(0.0)
        )
        garea = (g2 - g0) * (g3 - g1)
        iou = inter / (area + garea - inter)
        better = iou > mi
        mi = jnp.where(better, iou, mi)
        ga = jnp.where(better, g, ga)
        return mi, ga

    mi0 = jnp.full((_ROWS, 128), _NEG_INF, jnp.float32)
    ga0 = jnp.zeros((_ROWS, 128), jnp.int32)
    mi, ga = jax.lax.fori_loop(0, _G, iou_step, (mi0, ga0))

    lin = (jax.lax.broadcasted_iota(jnp.int32, (_ROWS, 128), 0) * 128
           + jax.lax.broadcasted_iota(jnp.int32, (_ROWS, 128), 1))
    mi = jnp.where(lin < _N, mi, _NEG_INF)
    kp = _orderkey(jnp.where(mi >= 0.5, mi, _NEG_INF))
    kn = _orderkey(jnp.where(mi < 0.5, mi, _NEG_INF))
    kp_ref[...] = kp
    kn_ref[...] = kn
    ga_ref[...] = ga

    one = jnp.ones((1, 1), jnp.int32)

    def t_step(_, carry):
        lo_p, hi_p, lo_n, hi_n = carry
        mid_p = lo_p + jax.lax.shift_right_logical(hi_p - lo_p, 1)
        mid_n = lo_n + jax.lax.shift_right_logical(hi_n - lo_n, 1)
        cnt_p = _sum11((kp > mid_p).astype(jnp.int32))
        cnt_n = _sum11((kn > mid_n).astype(jnp.int32))
        pred_p = cnt_p < _POS
        pred_n = cnt_n < _NEG
        hi_p = jnp.where(pred_p, mid_p, hi_p)
        lo_p = jnp.where(pred_p, lo_p, mid_p)
        hi_n = jnp.where(pred_n, mid_n, hi_n)
        lo_n = jnp.where(pred_n, lo_n, mid_n)
        return lo_p, hi_p, lo_n, hi_n

    lo0 = jnp.full((1, 1), _I32_MIN, jnp.int32)
    hi0 = jnp.full((1, 1), _I32_MAX, jnp.int32)
    _, tp, _, tn = jax.lax.fori_loop(0, 32, t_step, (lo0, hi0, lo0, hi0))

    need_p = _POS * one - _sum11((kp > tp).astype(jnp.int32))
    need_n = _NEG * one - _sum11((kn > tn).astype(jnp.int32))
    eq_p = kp == tp
    eq_n = kn == tn

    def x_step(_, carry):
        lo_p, hi_p, lo_n, hi_n = carry
        mid_p = jax.lax.shift_right_arithmetic(lo_p + hi_p, 1)
        mid_n = jax.lax.shift_right_arithmetic(lo_n + hi_n, 1)
        cnt_p = _sum11((eq_p & (lin < mid_p)).astype(jnp.int32))
        cnt_n = _sum11((eq_n & (lin < mid_n)).astype(jnp.int32))
        pred_p = cnt_p >= need_p
        pred_n = cnt_n >= need_n
        hi_p = jnp.where(pred_p, mid_p, hi_p)
        lo_p = jnp.where(pred_p, lo_p, mid_p)
        hi_n = jnp.where(pred_n, mid_n, hi_n)
        lo_n = jnp.where(pred_n, lo_n, mid_n)
        return lo_p, hi_p, lo_n, hi_n

    xlo0 = jnp.full((1, 1), -1, jnp.int32)
    xhi0 = jnp.full((1, 1), _NPAD, jnp.int32)
    _, xp, _, xn = jax.lax.fori_loop(0, 15, x_step, (xlo0, xhi0, xlo0, xhi0))

    iota_c = jax.lax.broadcasted_iota(jnp.int32, (1, 128), 1)
    slot_col = jax.lax.broadcasted_iota(jnp.int32, (128, 1), 0)
    slotf_col = slot_col.astype(jnp.float32)
    tri = jnp.where(
        jax.lax.broadcasted_iota(jnp.int32, (128, 128), 0)
        < jax.lax.broadcasted_iota(jnp.int32, (128, 128), 1),
        jnp.float32(1.0), jnp.float32(0.0))
    eye = jnp.where(
        jax.lax.broadcasted_iota(jnp.int32, (128, 128), 0)
        == jax.lax.broadcasted_iota(jnp.int32, (128, 128), 1),
        jnp.float32(1.0), jnp.float32(0.0))

    def key_splits(krow):
        hi = jax.lax.shift_right_arithmetic(krow, 16).astype(jnp.float32)
        lo = (krow & jnp.int32(0xFFFF)).astype(jnp.float32)
        return hi, lo

    def chunk_step(r, carry):
        acc, base_p, base_n = carry
        kp_row = kp_ref[pl.ds(r, 1), :]
        kn_row = kn_ref[pl.ds(r, 1), :]
        ga_row = ga_ref[pl.ds(r, 1), :].astype(jnp.float32)
        lin_row = iota_c + r * 128
        linf_row = lin_row.astype(jnp.float32)
        selp = (kp_row > tp) | ((kp_row == tp) & (lin_row < xp))
        seln = (kn_row > tn) | ((kn_row == tn) & (lin_row < xn))
        sp = jnp.where(selp, jnp.float32(1.0), jnp.float32(0.0))
        sn = jnp.where(seln, jnp.float32(1.0), jnp.float32(0.0))
        p_pos = base_p + _mm(sp, tri)
        p_neg = base_n + _mm(sn, tri) + jnp.float32(_POS)
        gp = jnp.where((p_pos == slotf_col) & selp, jnp.float32(1.0),
                       jnp.float32(0.0))
        gn = jnp.where((p_neg == slotf_col) & seln, jnp.float32(1.0),
                       jnp.float32(0.0))
        bchunk = boxes_tr_ref[:, pl.ds(r, 1), :].reshape(4, 128)
        khi_p, klo_p = key_splits(kp_row)
        khi_n, klo_n = key_splits(kn_row)
        xpk = jnp.concatenate([bchunk, ga_row, khi_p, klo_p, linf_row], axis=0)
        xnk = jnp.concatenate([bchunk, ga_row, khi_n, klo_n, linf_row], axis=0)
        acc = acc + jax.lax.dot_general(
            gp, xpk, (((1,), (1,)), ((), ())),
            preferred_element_type=jnp.float32,
            precision=jax.lax.Precision.HIGHEST)
        acc = acc + jax.lax.dot_general(
            gn, xnk, (((1,), (1,)), ((), ())),
            preferred_element_type=jnp.float32,
            precision=jax.lax.Precision.HIGHEST)
        base_p = base_p + jnp.sum(sp, axis=1, keepdims=True)
        base_n = base_n + jnp.sum(sn, axis=1, keepdims=True)
        return acc, base_p, base_n

    acc0 = jnp.zeros((_K, 8), jnp.float32)
    basef0 = jnp.zeros((1, 1), jnp.float32)
    acc, _, _ = jax.lax.fori_loop(0, _ROWS, chunk_step, (acc0, basef0, basef0))

    accT = jax.lax.dot_general(acc, eye, _DT,
                               preferred_element_type=jnp.float32,
                               precision=jax.lax.Precision.HIGHEST)
    key_col = (acc[:, 5:6].astype(jnp.int32) * 65536
               + acc[:, 6:7].astype(jnp.int32))
    idx_col = acc[:, 7:8].astype(jnp.int32)
    key_row = (accT[5:6, :].astype(jnp.int32) * 65536
               + accT[6:7, :].astype(jnp.int32))
    idx_row = accT[7:8, :].astype(jnp.int32)
    is_pos_col = (slot_col < _POS).astype(jnp.int32)
    is_pos_row = (iota_c < _POS).astype(jnp.int32)
    higher = (key_col > key_row) | ((key_col == key_row) & (idx_col < idx_row))
    same = is_pos_col == is_pos_row
    rank_row = jnp.sum(
        jnp.where(higher & same, jnp.float32(1.0), jnp.float32(0.0)),
        axis=0, keepdims=True)
    out_slot_row = rank_row + jnp.where(
        is_pos_row == 1, jnp.float32(0.0), jnp.float32(_POS))
    perm = jnp.where(out_slot_row == slotf_col, jnp.float32(1.0),
                     jnp.float32(0.0))
    final = _mm(perm, acc)

    roi = final[:, 0:4]
    ga_f = final[:, 4:5].astype(jnp.int32)
    key_f = (final[:, 5:6].astype(jnp.int32) * 65536
             + final[:, 6:7].astype(jnp.int32))
    score = jax.lax.bitcast_convert_type(
        jnp.where(key_f < 0, key_f ^ jnp.int32(0x7FFFFFFF), key_f),
        jnp.float32)
    iota64 = jax.lax.broadcasted_iota(jnp.int32, (1, _G), 1)
    onehot_gt = jnp.where(ga_f == iota64, jnp.float32(1.0), jnp.float32(0.0))
    gtn = _mm(onehot_gt, gt4_ref[...])

    h = roi[:, 2:3] - roi[:, 0:1]
    w = roi[:, 3:4] - roi[:, 1:2]
    dy = (gtn[:, 2:3] + gtn[:, 0:1] - roi[:, 2:3] - roi[:, 0:1]) / 2.0 / h
    dx = (gtn[:, 3:4] + gtn[:, 2:3] - roi[:, 3:4] - roi[:, 2:3]) / 2.0 / w
    dh = jnp.log(jnp.maximum(h - gtn[:, 2:3] + gtn[:, 0:1], jnp.float32(1e-6)))
    dw = jnp.log(jnp.maximum(w - gtn[:, 3:4] + gtn[:, 1:2], jnp.float32(1e-6)))

    roi_ref[...] = roi
    gtn_ref[...] = gtn
    label_ref[...] = (score >= 0.5).astype(jnp.int32)
    loc_ref[...] = jnp.concatenate([dy, dx, dh, dw], axis=1)


@jax.jit
def kernel(boxes, gt_bboxes):
    boxes_p = jnp.pad(boxes, ((0, _NPAD - _N), (0, 0)))
    boxes_tr = boxes_p.T.reshape(4, _ROWS, 128)
    gt_t = gt_bboxes.T

    roi, gtn, label, loc = pl.pallas_call(
        _body,
        out_shape=[
            jax.ShapeDtypeStruct((_K, 4), jnp.float32),
            jax.ShapeDtypeStruct((_K, 4), jnp.float32),
            jax.ShapeDtypeStruct((_K, 1), jnp.int32),
            jax.ShapeDtypeStruct((_K, 4), jnp.float32),
        ],
        in_specs=[
            pl.BlockSpec(memory_space=pltpu.VMEM),
            pl.BlockSpec(memory_space=pltpu.SMEM),
            pl.BlockSpec(memory_space=pltpu.VMEM),
        ],
        scratch_shapes=[
            pltpu.VMEM((_ROWS, 128), jnp.int32),
            pltpu.VMEM((_ROWS, 128), jnp.int32),
            pltpu.VMEM((_ROWS, 128), jnp.int32),
        ],
    )(boxes_tr, gt_t, gt_bboxes)

    return roi, gtn, label.reshape(_K), loc


# speedup vs baseline: 2.4370x; 2.4370x over previous
import jax
import jax.numpy as jnp
import numpy as np
from jax.experimental import pallas as pl
from jax.experimental.pallas import tpu as pltpu

_N = 20000
_NPAD = 20480
_ROWS = _NPAD // 128
_G = 64
_POS = 32
_NEG = 96
_K = _POS + _NEG
_NPL = 30

_NEG_INF = np.float32(-np.inf)
_I32_MIN = np.int32(-(2 ** 31))
_I32_MAX = np.int32(2 ** 31 - 1)

_DN = (((1,), (0,)), ((), ()))
_DT = (((0,), (0,)), ((), ()))
_DL = (((1,), (1,)), ((), ()))


def _orderkey(x):
    b = jax.lax.bitcast_convert_type(x, jnp.int32)
    return jnp.where(b < 0, b ^ jnp.int32(0x7FFFFFFF), b)


def _sum11(x):
    return jnp.sum(jnp.sum(x, axis=0, keepdims=True), axis=1, keepdims=True)


def _planes4(v_i32):
    p0 = jax.lax.shift_right_arithmetic(v_i32, 24).astype(jnp.float32)
    p1 = (jax.lax.shift_right_arithmetic(v_i32, 16)
          & jnp.int32(0xFF)).astype(jnp.float32)
    p2 = (jax.lax.shift_right_arithmetic(v_i32, 8)
          & jnp.int32(0xFF)).astype(jnp.float32)
    p3 = (v_i32 & jnp.int32(0xFF)).astype(jnp.float32)
    return p0, p1, p2, p3


def _recon4(p0, p1, p2, p3):
    return (p0.astype(jnp.int32) * jnp.int32(1 << 24)
            + p1.astype(jnp.int32) * jnp.int32(1 << 16)
            + p2.astype(jnp.int32) * jnp.int32(1 << 8)
            + p3.astype(jnp.int32))


def _body(boxes_tr_ref, gt_ref, gt4_ref,
          roi_ref, gtn_ref, label_ref, loc_ref, x_ref):
    b0 = boxes_tr_ref[0]
    b1 = boxes_tr_ref[1]
    b2 = boxes_tr_ref[2]
    b3 = boxes_tr_ref[3]
    area = (b2 - b0) * (b3 - b1)

    def iou_step(g, carry):
        mi, ga = carry
        g0 = gt_ref[0, g]
        g1 = gt_ref[1, g]
        g2 = gt_ref[2, g]
        g3 = gt_ref[3, g]
        ty = jnp.maximum(b0, g0)
        tx = jnp.maximum(b1, g1)
        by = jnp.minimum(b2, g2)
        bx = jnp.minimum(b3, g3)
        inter = ((by - ty) * (bx - tx)) * jnp.where(
            (ty < by) & (tx < bx), jnp.float32(1.0), jnp.float32(0.0)
        )
        garea = (g2 - g0) * (g3 - g1)
        iou = inter / (area + garea - inter)
        better = iou > mi
        mi = jnp.where(better, iou, mi)
        ga = jnp.where(better, g, ga)
        return mi, ga

    mi0 = jnp.full((_ROWS, 128), _NEG_INF, jnp.float32)
    ga0 = jnp.zeros((_ROWS, 128), jnp.int32)
    mi, ga = jax.lax.fori_loop(0, _G, iou_step, (mi0, ga0))

    lin = (jax.lax.broadcasted_iota(jnp.int32, (_ROWS, 128), 0) * 128
           + jax.lax.broadcasted_iota(jnp.int32, (_ROWS, 128), 1))
    mi = jnp.where(lin < _N, mi, _NEG_INF)
    kp = _orderkey(jnp.where(mi >= 0.5, mi, _NEG_INF))
    kn = _orderkey(jnp.where(mi < 0.5, mi, _NEG_INF))

    one = jnp.ones((1, 1), jnp.int32)

    def t_step(_, carry):
        lo_p, hi_p, lo_n, hi_n = carry
        mid_p = lo_p + jax.lax.shift_right_logical(hi_p - lo_p, 1)
        mid_n = lo_n + jax.lax.shift_right_logical(hi_n - lo_n, 1)
        cnt_p = _sum11((kp > mid_p).astype(jnp.int32))
        cnt_n = _sum11((kn > mid_n).astype(jnp.int32))
        pred_p = cnt_p < _POS
        pred_n = cnt_n < _NEG
        hi_p = jnp.where(pred_p, mid_p, hi_p)
        lo_p = jnp.where(pred_p, lo_p, mid_p)
        hi_n = jnp.where(pred_n, mid_n, hi_n)
        lo_n = jnp.where(pred_n, lo_n, mid_n)
        return lo_p, hi_p, lo_n, hi_n

    lo0 = jnp.full((1, 1), _I32_MIN, jnp.int32)
    hi0 = jnp.full((1, 1), _I32_MAX, jnp.int32)
    _, tp, _, tn = jax.lax.fori_loop(0, 32, t_step, (lo0, hi0, lo0, hi0))

    need_p = _POS * one - _sum11((kp > tp).astype(jnp.int32))
    need_n = _NEG * one - _sum11((kn > tn).astype(jnp.int32))
    eq_p = kp == tp
    eq_n = kn == tn

    def x_step(_, carry):
        lo_p, hi_p, lo_n, hi_n = carry
        mid_p = jax.lax.shift_right_arithmetic(lo_p + hi_p, 1)
        mid_n = jax.lax.shift_right_arithmetic(lo_n + hi_n, 1)
        cnt_p = _sum11((eq_p & (lin < mid_p)).astype(jnp.int32))
        cnt_n = _sum11((eq_n & (lin < mid_n)).astype(jnp.int32))
        pred_p = cnt_p >= need_p
        pred_n = cnt_n >= need_n
        hi_p = jnp.where(pred_p, mid_p, hi_p)
        lo_p = jnp.where(pred_p, lo_p, mid_p)
        hi_n = jnp.where(pred_n, mid_n, hi_n)
        lo_n = jnp.where(pred_n, lo_n, mid_n)
        return lo_p, hi_p, lo_n, hi_n

    xlo0 = jnp.full((1, 1), -1, jnp.int32)
    xhi0 = jnp.full((1, 1), _NPAD, jnp.int32)
    _, xp, _, xn = jax.lax.fori_loop(0, 15, x_step, (xlo0, xhi0, xlo0, xhi0))

    selp = (kp > tp) | (eq_p & (lin < xp))
    seln = (kn > tn) | (eq_n & (lin < xn))
    sp = jnp.where(selp, jnp.float32(1.0), jnp.float32(0.0))
    sn = jnp.where(seln, jnp.float32(1.0), jnp.float32(0.0))
    tri = jnp.where(
        jax.lax.broadcasted_iota(jnp.int32, (128, 128), 0)
        < jax.lax.broadcasted_iota(jnp.int32, (128, 128), 1),
        jnp.float32(1.0), jnp.float32(0.0))
    tri_r = jnp.where(
        jax.lax.broadcasted_iota(jnp.int32, (_ROWS, _ROWS), 1)
        < jax.lax.broadcasted_iota(jnp.int32, (_ROWS, _ROWS), 0),
        jnp.float32(1.0), jnp.float32(0.0))
    pref_p = jax.lax.dot_general(sp, tri, _DN,
                                 preferred_element_type=jnp.float32)
    pref_n = jax.lax.dot_general(sn, tri, _DN,
                                 preferred_element_type=jnp.float32)
    rs_p = jnp.sum(sp, axis=1, keepdims=True)
    rs_n = jnp.sum(sn, axis=1, keepdims=True)
    base_p = jax.lax.dot_general(tri_r, rs_p, _DN,
                                 preferred_element_type=jnp.float32)
    base_n = jax.lax.dot_general(tri_r, rs_n, _DN,
                                 preferred_element_type=jnp.float32)
    slot_p = jnp.where(selp, pref_p + base_p, jnp.float32(255.0))
    slot_n = jnp.where(seln, pref_n + base_n + jnp.float32(_POS),
                       jnp.float32(255.0))

    gaf = ga.astype(jnp.float32)
    lin_hi = jax.lax.shift_right_arithmetic(lin, 8).astype(jnp.float32)
    lin_lo = (lin & jnp.int32(0xFF)).astype(jnp.float32)
    planes = []
    for coord in (b0, b1, b2, b3):
        planes.extend(_planes4(jax.lax.bitcast_convert_type(coord, jnp.int32)))
    planes.append(gaf)
    planes.extend([lin_hi, lin_lo])
    planes.extend(_planes4(kp))
    planes.extend(_planes4(kn))
    planes.extend([slot_p, slot_n, jnp.zeros((_ROWS, 128), jnp.float32)])
    for p_i, pv in enumerate(planes):
        x_ref[:, pl.ds(p_i, 1), :] = pv.reshape(_ROWS, 1, 128)

    slotf_col = jax.lax.broadcasted_iota(jnp.int32, (128, 1), 0).astype(
        jnp.float32)

    def chunk_step(r, acc):
        xc = x_ref[pl.ds(r, 1), :, :].reshape(_NPL, 128)
        gp = jnp.where(xc[27:28, :] == slotf_col, jnp.float32(1.0),
                       jnp.float32(0.0))
        gn = jnp.where(xc[28:29, :] == slotf_col, jnp.float32(1.0),
                       jnp.float32(0.0))
        return acc + jax.lax.dot_general(
            gp + gn, xc, _DL, preferred_element_type=jnp.float32)

    acc0 = jnp.zeros((_K, _NPL), jnp.float32)
    acc = jax.lax.fori_loop(0, _ROWS, chunk_step, acc0)

    eye = jnp.where(
        jax.lax.broadcasted_iota(jnp.int32, (128, 128), 0)
        == jax.lax.broadcasted_iota(jnp.int32, (128, 128), 1),
        jnp.float32(1.0), jnp.float32(0.0))
    acc_t = jax.lax.dot_general(acc, eye, _DT,
                                preferred_element_type=jnp.float32)
    slot_col = jax.lax.broadcasted_iota(jnp.int32, (128, 1), 0)
    iota_c = jax.lax.broadcasted_iota(jnp.int32, (1, 128), 1)
    is_pos_col = slot_col < _POS
    is_pos_row = iota_c < _POS

    kp_col = _recon4(acc[:, 19:20], acc[:, 20:21], acc[:, 21:22],
                     acc[:, 22:23])
    kn_col = _recon4(acc[:, 23:24], acc[:, 24:25], acc[:, 25:26],
                     acc[:, 26:27])
    key_col = jnp.where(is_pos_col, kp_col, kn_col)
    kp_row = _recon4(acc_t[19:20, :], acc_t[20:21, :], acc_t[21:22, :],
                     acc_t[22:23, :])
    kn_row = _recon4(acc_t[23:24, :], acc_t[24:25, :], acc_t[25:26, :],
                     acc_t[26:27, :])
    key_row = jnp.where(is_pos_row, kp_row, kn_row)
    idx_col = (acc[:, 17:18].astype(jnp.int32) * 256
               + acc[:, 18:19].astype(jnp.int32))
    idx_row = (acc_t[17:18, :].astype(jnp.int32) * 256
               + acc_t[18:19, :].astype(jnp.int32))

    higher = (key_col > key_row) | ((key_col == key_row) & (idx_col < idx_row))
    same = is_pos_col == is_pos_row
    rank_row = jnp.sum(
        jnp.where(higher & same, jnp.float32(1.0), jnp.float32(0.0)),
        axis=0, keepdims=True)
    out_slot_row = rank_row + jnp.where(
        is_pos_row, jnp.float32(0.0), jnp.float32(_POS))
    perm = jnp.where(out_slot_row == slotf_col, jnp.float32(1.0),
                     jnp.float32(0.0))
    final = jax.lax.dot_general(perm, acc, _DN,
                                preferred_element_type=jnp.float32)

    coords = [
        jax.lax.bitcast_convert_type(
            _recon4(final[:, 4 * c + 0:4 * c + 1], final[:, 4 * c + 1:4 * c + 2],
                    final[:, 4 * c + 2:4 * c + 3], final[:, 4 * c + 3:4 * c + 4]),
            jnp.float32)
        for c in range(4)
    ]
    roi = jnp.concatenate(coords, axis=1)
    ga_f = final[:, 16:17].astype(jnp.int32)
    key_f = jnp.where(is_pos_col,
                      _recon4(final[:, 19:20], final[:, 20:21],
                              final[:, 21:22], final[:, 22:23]),
                      _recon4(final[:, 23:24], final[:, 24:25],
                              final[:, 25:26], final[:, 26:27]))
    score = jax.lax.bitcast_convert_type(
        jnp.where(key_f < 0, key_f ^ jnp.int32(0x7FFFFFFF), key_f),
        jnp.float32)
    iota64 = jax.lax.broadcasted_iota(jnp.int32, (1, _G), 1)
    onehot_gt = jnp.where(ga_f == iota64, jnp.float32(1.0), jnp.float32(0.0))
    gtn = jax.lax.dot_general(onehot_gt, gt4_ref[...], _DN,
                              preferred_element_type=jnp.float32,
                              precision=jax.lax.Precision.HIGHEST)

    h = roi[:, 2:3] - roi[:, 0:1]
    w = roi[:, 3:4] - roi[:, 1:2]
    dy = (gtn[:, 2:3] + gtn[:, 0:1] - roi[:, 2:3] - roi[:, 0:1]) / 2.0 / h
    dx = (gtn[:, 3:4] + gtn[:, 2:3] - roi[:, 3:4] - roi[:, 2:3]) / 2.0 / w
    dh = jnp.log(jnp.maximum(h - gtn[:, 2:3] + gtn[:, 0:1], jnp.float32(1e-6)))
    dw = jnp.log(jnp.maximum(w - gtn[:, 3:4] + gtn[:, 1:2], jnp.float32(1e-6)))

    roi_ref[...] = roi
    gtn_ref[...] = gtn
    label_ref[...] = (score >= 0.5).astype(jnp.int32)
    loc_ref[...] = jnp.concatenate([dy, dx, dh, dw], axis=1)


@jax.jit
def kernel(boxes, gt_bboxes):
    boxes_p = jnp.pad(boxes, ((0, _NPAD - _N), (0, 0)))
    boxes_tr = boxes_p.T.reshape(4, _ROWS, 128)
    gt_t = gt_bboxes.T

    roi, gtn, label, loc = pl.pallas_call(
        _body,
        out_shape=[
            jax.ShapeDtypeStruct((_K, 4), jnp.float32),
            jax.ShapeDtypeStruct((_K, 4), jnp.float32),
            jax.ShapeDtypeStruct((_K, 1), jnp.int32),
            jax.ShapeDtypeStruct((_K, 4), jnp.float32),
        ],
        in_specs=[
            pl.BlockSpec(memory_space=pltpu.VMEM),
            pl.BlockSpec(memory_space=pltpu.SMEM),
            pl.BlockSpec(memory_space=pltpu.VMEM),
        ],
        scratch_shapes=[
            pltpu.VMEM((_ROWS, _NPL, 128), jnp.float32),
        ],
    )(boxes_tr, gt_t, gt_bboxes)

    return roi, gtn, label.reshape(_K), loc
